# baseline (speedup 1.0000x reference)
import jax
import jax.numpy as jnp
from jax import lax
from jax.experimental import pallas as pl
from jax.experimental.pallas import tpu as pltpu
from jax.experimental.pallas import tpu_sc as plsc

SEG = 1000
PAD_SEG = 1024
EMB = 64
TBL_WORDS = PAD_SEG * EMB
SCALE = float(EMB) ** 0.5
NC = 2
NS = 16
NW = NC * NS


def _body(wordT_hbm, tbl_hbm, outT_hbm, idxT_v, tbl_v, obuf, osem):
    c = lax.axis_index("c")
    sub = lax.axis_index("s")
    w = c * NS + sub
    seq, batch = wordT_hbm.shape
    bpw = batch // NW
    b0 = w * bpw

    pltpu.sync_copy(wordT_hbm.at[:, pl.ds(b0, bpw)], idxT_v)
    pltpu.sync_copy(tbl_hbm, tbl_v)

    def scale_body(i, carry):
        for j in range(8):
            tbl_v[pl.ds(i * 128 + j * 16, 16)] = (
                tbl_v[pl.ds(i * 128 + j * 16, 16)] * SCALE)
        return carry

    lax.fori_loop(0, TBL_WORDS // 128, scale_body, 0)

    def compute(s, par):
        for bg in range(bpw // 16):
            idx16 = idxT_v[s, pl.ds(bg * 16, 16)]
            ids = idx16 * EMB
            for e in range(EMB):
                v = plsc.load_gather(tbl_v, [ids + e])
                obuf[par, e, pl.ds(bg * 16, 16)] = v
        pltpu.async_copy(obuf.at[par],
                         outT_hbm.at[s].at[:, pl.ds(b0, bpw)],
                         osem.at[par])

    def wait_out(s, par):
        pltpu.make_async_copy(obuf.at[par],
                              outT_hbm.at[s].at[:, pl.ds(b0, bpw)],
                              osem.at[par]).wait()

    compute(0, 0)
    compute(1, 1)

    def loop_body(t, carry):
        s = t * 2
        wait_out(s, 0)
        compute(s + 2, 0)
        wait_out(s + 1, 1)
        compute(s + 3, 1)
        return carry

    lax.fori_loop(0, seq // 2 - 1, loop_body, 0)
    wait_out(seq - 2, 0)
    wait_out(seq - 1, 1)


def _make_kernel(batch, seq):
    mesh = plsc.VectorSubcoreMesh(core_axis_name="c", subcore_axis_name="s")
    bpw = batch // NW
    return pl.kernel(
        _body,
        mesh=mesh,
        out_type=jax.ShapeDtypeStruct((seq, EMB, batch), jnp.float32),
        scratch_types=[
            pltpu.VMEM((seq, bpw), jnp.int32),
            pltpu.VMEM((TBL_WORDS,), jnp.float32),
            pltpu.VMEM((2, EMB, bpw), jnp.float32),
            pltpu.SemaphoreType.DMA((2,)),
        ],
        compiler_params=pltpu.CompilerParams(needs_layout_passes=False),
    )


@jax.jit
def kernel(word, seg_embedding_weight):
    batch, seq = word.shape
    wordT = word.T.astype(jnp.int32)
    tbl_flat = jnp.zeros((PAD_SEG, EMB), jnp.float32).at[:SEG, :].set(
        seg_embedding_weight).reshape(-1)
    outT = _make_kernel(batch, seq)(wordT, tbl_flat)
    return jnp.transpose(outT, (2, 0, 1))


# speedup vs baseline: 1.8465x; 1.8465x over previous
import jax
import jax.numpy as jnp
from jax import lax
from jax.experimental import pallas as pl
from jax.experimental.pallas import tpu as pltpu
from jax.experimental.pallas import tpu_sc as plsc

SEG = 1000
PAD_SEG = 1024
EMB = 64
ROWSTRIDE = 65
TBL_WORDS = PAD_SEG * ROWSTRIDE
SCALE = float(EMB) ** 0.5
NC = 2
NS = 16
NW = NC * NS


def _body(wordT_hbm, tbl_hbm, outT_hbm, idxT_v, tbl_v, obuf, osem):
    c = lax.axis_index("c")
    sub = lax.axis_index("s")
    w = c * NS + sub
    seq, batch = wordT_hbm.shape
    bpw = batch // NW
    b0 = w * bpw

    pltpu.sync_copy(wordT_hbm.at[:, pl.ds(b0, bpw)], idxT_v)
    pltpu.sync_copy(tbl_hbm, tbl_v)

    def scale_body(i, carry):
        for j in range(8):
            tbl_v[pl.ds(i * 128 + j * 16, 16)] = (
                tbl_v[pl.ds(i * 128 + j * 16, 16)] * SCALE)
        return carry

    lax.fori_loop(0, TBL_WORDS // 128, scale_body, 0)

    def compute(s, par):
        for bg in range(bpw // 16):
            idx16 = idxT_v[s, pl.ds(bg * 16, 16)]
            ids = idx16 * ROWSTRIDE
            for e in range(EMB):
                v = plsc.load_gather(tbl_v, [ids + e])
                obuf[par, e, pl.ds(bg * 16, 16)] = v
        pltpu.async_copy(obuf.at[par],
                         outT_hbm.at[s].at[:, pl.ds(b0, bpw)],
                         osem.at[par])

    def wait_out(s, par):
        pltpu.make_async_copy(obuf.at[par],
                              outT_hbm.at[s].at[:, pl.ds(b0, bpw)],
                              osem.at[par]).wait()

    compute(0, 0)
    compute(1, 1)

    def loop_body(t, carry):
        s = t * 2
        wait_out(s, 0)
        compute(s + 2, 0)
        wait_out(s + 1, 1)
        compute(s + 3, 1)
        return carry

    lax.fori_loop(0, seq // 2 - 1, loop_body, 0)
    wait_out(seq - 2, 0)
    wait_out(seq - 1, 1)


def _make_kernel(batch, seq):
    mesh = plsc.VectorSubcoreMesh(core_axis_name="c", subcore_axis_name="s")
    bpw = batch // NW
    return pl.kernel(
        _body,
        mesh=mesh,
        out_type=jax.ShapeDtypeStruct((seq, EMB, batch), jnp.float32),
        scratch_types=[
            pltpu.VMEM((seq, bpw), jnp.int32),
            pltpu.VMEM((TBL_WORDS,), jnp.float32),
            pltpu.VMEM((2, EMB, bpw), jnp.float32),
            pltpu.SemaphoreType.DMA((2,)),
        ],
        compiler_params=pltpu.CompilerParams(needs_layout_passes=False),
    )


@jax.jit
def kernel(word, seg_embedding_weight):
    batch, seq = word.shape
    wordT = word.T.astype(jnp.int32)
    tbl_flat = jnp.zeros((PAD_SEG, ROWSTRIDE), jnp.float32).at[:SEG, :EMB].set(
        seg_embedding_weight).reshape(-1)
    outT = _make_kernel(batch, seq)(wordT, tbl_flat)
    return jnp.transpose(outT, (2, 0, 1))


# speedup vs baseline: 2.6166x; 1.4171x over previous
import jax
import jax.numpy as jnp
from jax import lax
from jax.experimental import pallas as pl
from jax.experimental.pallas import tpu as pltpu
from jax.experimental.pallas import tpu_sc as plsc
from jax.experimental.layout import Layout, with_layout_constraint

SEG = 1000
PAD_SEG = 1024
EMB = 64
WROW = 128
SCALE = float(EMB) ** 0.5
NC = 2
NS = 16
NW = NC * NS
CHUNK = 256
NBUF = 2
ROWS_PER_SUB = PAD_SEG // NS


def _body(word_hbm, table_hbm, out_hbm, scaled_hbm, idx_v, rows_v, tbl_v,
          gsem, osem):
    c = lax.axis_index("c")
    s = lax.axis_index("s")
    wid = c * NS + s

    row0 = s * ROWS_PER_SUB
    pltpu.sync_copy(table_hbm.at[pl.ds(row0, ROWS_PER_SUB)], tbl_v)

    def scale_row(r, carry):
        for j in range(EMB // 16):
            tbl_v[r, pl.ds(j * 16, 16)] = tbl_v[r, pl.ds(j * 16, 16)] * SCALE
        return carry

    lax.fori_loop(0, ROWS_PER_SUB, scale_row, 0)
    pltpu.sync_copy(tbl_v, scaled_hbm.at[c].at[pl.ds(row0, ROWS_PER_SUB)])
    plsc.subcore_barrier()

    n_idx = word_hbm.shape[0]
    per_w = n_idx // NW
    n_chunks = per_w // CHUNK
    n_super = n_chunks // NBUF
    base = wid * per_w
    pltpu.sync_copy(word_hbm.at[pl.ds(base, per_w)], idx_v)

    def start_gather(g, b):
        idx_slice = idx_v.at[pl.ds(g * CHUNK, CHUNK)]
        pltpu.async_copy(scaled_hbm.at[c].at[idx_slice],
                         rows_v.at[b], gsem.at[b])

    def wait_gather(b):
        pltpu.make_async_copy(scaled_hbm.at[c].at[idx_v.at[pl.ds(0, CHUNK)]],
                              rows_v.at[b], gsem.at[b]).wait()

    def start_out(g, b):
        pltpu.async_copy(rows_v.at[b],
                         out_hbm.at[pl.ds(base + g * CHUNK, CHUNK)],
                         osem.at[b])

    def wait_out(g, b):
        pltpu.make_async_copy(rows_v.at[b],
                              out_hbm.at[pl.ds(base + g * CHUNK, CHUNK)],
                              osem.at[b]).wait()

    for b in range(NBUF):
        start_gather(b, b)

    def super_body(t, carry):
        g0 = t * NBUF
        for b in range(NBUF):
            wait_gather(b)
            start_out(g0 + b, b)
        for b in range(NBUF):
            wait_out(g0 + b, b)
            start_gather(g0 + NBUF + b, b)
        return carry

    lax.fori_loop(0, n_super - 1, super_body, 0)

    g0 = (n_super - 1) * NBUF
    for b in range(NBUF):
        wait_gather(b)
        start_out(g0 + b, b)
    for b in range(NBUF):
        wait_out(g0 + b, b)


def _make_kernel(n_idx):
    mesh = plsc.VectorSubcoreMesh(core_axis_name="c", subcore_axis_name="s")
    per_w = n_idx // NW
    return pl.kernel(
        _body,
        mesh=mesh,
        out_type=jax.ShapeDtypeStruct((n_idx, EMB), jnp.float32),
        scratch_types=[
            pltpu.HBM((NC, PAD_SEG, EMB), jnp.float32),
            pltpu.VMEM((per_w,), jnp.int32),
            pltpu.VMEM((NBUF, CHUNK, EMB), jnp.float32),
            pltpu.VMEM((ROWS_PER_SUB, EMB), jnp.float32),
            pltpu.SemaphoreType.DMA((NBUF,)),
            pltpu.SemaphoreType.DMA((NBUF,)),
        ],
    )


@jax.jit
def kernel(word, seg_embedding_weight):
    batch, seq = word.shape
    word_flat = word.reshape(-1).astype(jnp.int32)
    table_pad = jnp.zeros((PAD_SEG, EMB), jnp.float32).at[:SEG, :].set(
        seg_embedding_weight)
    out = _make_kernel(batch * seq)(word_flat, table_pad)
    out = out.reshape(batch, seq, EMB)
    return with_layout_constraint(out, Layout(major_to_minor=(0, 1, 2)))


# speedup vs baseline: 2.6252x; 1.0033x over previous
import jax
import jax.numpy as jnp
from jax import lax
from jax.experimental import pallas as pl
from jax.experimental.pallas import tpu as pltpu
from jax.experimental.pallas import tpu_sc as plsc
from jax.experimental.layout import Layout, with_layout_constraint

SEG = 1000
PAD_SEG = 1024
EMB = 64
WROW = 128
SCALE = float(EMB) ** 0.5
NC = 2
NS = 16
NW = NC * NS
CHUNK = 128
NBUF = 5
ROWS_PER_SUB = PAD_SEG // NS


def _body(word_hbm, table_hbm, out_hbm, scaled_hbm, idx_v, rows_v, tbl_v,
          gsem, osem):
    c = lax.axis_index("c")
    s = lax.axis_index("s")
    wid = c * NS + s

    row0 = s * ROWS_PER_SUB
    pltpu.sync_copy(table_hbm.at[pl.ds(row0, ROWS_PER_SUB)], tbl_v)

    def scale_row(r, carry):
        for j in range(EMB // 16):
            tbl_v[r, pl.ds(j * 16, 16)] = tbl_v[r, pl.ds(j * 16, 16)] * SCALE
        return carry

    lax.fori_loop(0, ROWS_PER_SUB, scale_row, 0)
    pltpu.sync_copy(tbl_v, scaled_hbm.at[c].at[pl.ds(row0, ROWS_PER_SUB)])
    plsc.subcore_barrier()

    n_idx = word_hbm.shape[0]
    per_w = n_idx // NW
    n_chunks = per_w // CHUNK
    n_super = n_chunks // NBUF
    base = wid * per_w
    pltpu.sync_copy(word_hbm.at[pl.ds(base, per_w)], idx_v)

    def start_gather(g, b):
        idx_slice = idx_v.at[pl.ds(g * CHUNK, CHUNK)]
        pltpu.async_copy(scaled_hbm.at[c].at[idx_slice],
                         rows_v.at[b], gsem.at[b])

    def wait_gather(b):
        pltpu.make_async_copy(scaled_hbm.at[c].at[idx_v.at[pl.ds(0, CHUNK)]],
                              rows_v.at[b], gsem.at[b]).wait()

    def start_out(g, b):
        pltpu.async_copy(rows_v.at[b],
                         out_hbm.at[pl.ds(base + g * CHUNK, CHUNK)],
                         osem.at[b])

    def wait_out(g, b):
        pltpu.make_async_copy(rows_v.at[b],
                              out_hbm.at[pl.ds(base + g * CHUNK, CHUNK)],
                              osem.at[b]).wait()

    for b in range(NBUF):
        start_gather(b, b)

    def super_body(t, carry):
        g0 = t * NBUF
        for b in range(NBUF):
            wait_gather(b)
            start_out(g0 + b, b)
        for b in range(NBUF):
            wait_out(g0 + b, b)
            start_gather(g0 + NBUF + b, b)
        return carry

    lax.fori_loop(0, n_super - 1, super_body, 0)

    g0 = (n_super - 1) * NBUF
    for b in range(NBUF):
        wait_gather(b)
        start_out(g0 + b, b)
    for b in range(NBUF):
        wait_out(g0 + b, b)


def _make_kernel(n_idx):
    mesh = plsc.VectorSubcoreMesh(core_axis_name="c", subcore_axis_name="s")
    per_w = n_idx // NW
    return pl.kernel(
        _body,
        mesh=mesh,
        out_type=jax.ShapeDtypeStruct((n_idx, EMB), jnp.float32),
        scratch_types=[
            pltpu.HBM((NC, PAD_SEG, EMB), jnp.float32),
            pltpu.VMEM((per_w,), jnp.int32),
            pltpu.VMEM((NBUF, CHUNK, EMB), jnp.float32),
            pltpu.VMEM((ROWS_PER_SUB, EMB), jnp.float32),
            pltpu.SemaphoreType.DMA((NBUF,)),
            pltpu.SemaphoreType.DMA((NBUF,)),
        ],
    )


@jax.jit
def kernel(word, seg_embedding_weight):
    batch, seq = word.shape
    word_flat = word.reshape(-1).astype(jnp.int32)
    table_pad = jnp.zeros((PAD_SEG, EMB), jnp.float32).at[:SEG, :].set(
        seg_embedding_weight)
    out = _make_kernel(batch * seq)(word_flat, table_pad)
    out = out.reshape(batch, seq, EMB)
    return with_layout_constraint(out, Layout(major_to_minor=(0, 1, 2)))


# speedup vs baseline: 3.7882x; 1.4430x over previous
import jax
import jax.numpy as jnp
from jax import lax
from jax.experimental import pallas as pl
from jax.experimental.pallas import tpu as pltpu
from jax.experimental.pallas import tpu_sc as plsc
from jax.experimental.layout import Layout, with_layout_constraint

SEG = 1000
PAD_SEG = 1024
EMB = 64
WROW = 128
SCALE = float(EMB) ** 0.5
NC = 2
NS = 16
NW = NC * NS
CHUNK = 128
NBUF = 4
ROWS_PER_SUB = PAD_SEG // NS


def _body(word_hbm, table_hbm, out_hbm, scaled_hbm, idx_v, rows_v, tbl_v,
          gsem, osem):
    c = lax.axis_index("c")
    s = lax.axis_index("s")
    wid = c * NS + s

    row0 = s * ROWS_PER_SUB
    pltpu.sync_copy(table_hbm.at[pl.ds(row0, ROWS_PER_SUB)], tbl_v)

    def scale_row(r, carry):
        for j in range(EMB // 16):
            tbl_v[r, pl.ds(j * 16, 16)] = tbl_v[r, pl.ds(j * 16, 16)] * SCALE
        return carry

    lax.fori_loop(0, ROWS_PER_SUB, scale_row, 0)
    pltpu.sync_copy(tbl_v, scaled_hbm.at[pl.ds(row0, ROWS_PER_SUB)])
    plsc.subcore_barrier()

    n_idx = word_hbm.shape[0]
    per_w = n_idx // NW
    n_chunks = per_w // CHUNK
    n_super = n_chunks // NBUF
    base = wid * per_w
    pltpu.sync_copy(word_hbm.at[pl.ds(base, per_w)], idx_v)

    def start_gather(g, b):
        idx_slice = idx_v.at[pl.ds(g * CHUNK, CHUNK)]
        pltpu.async_copy(scaled_hbm.at[idx_slice],
                         rows_v.at[b], gsem.at[b])

    def wait_gather(b):
        pltpu.make_async_copy(scaled_hbm.at[idx_v.at[pl.ds(0, CHUNK)]],
                              rows_v.at[b], gsem.at[b]).wait()

    def start_out(g, b):
        pltpu.async_copy(rows_v.at[b],
                         out_hbm.at[pl.ds(base + g * CHUNK, CHUNK)],
                         osem.at[b])

    def wait_out(g, b):
        pltpu.make_async_copy(rows_v.at[b],
                              out_hbm.at[pl.ds(base + g * CHUNK, CHUNK)],
                              osem.at[b]).wait()

    for b in range(NBUF):
        start_gather(b, b)

    def super_body(t, carry):
        g0 = t * NBUF
        for b in range(NBUF):
            wait_gather(b)
            start_out(g0 + b, b)
        for b in range(NBUF):
            wait_out(g0 + b, b)
            start_gather(g0 + NBUF + b, b)
        return carry

    lax.fori_loop(0, n_super - 1, super_body, 0)

    g0 = (n_super - 1) * NBUF
    for b in range(NBUF):
        wait_gather(b)
        start_out(g0 + b, b)
    for b in range(NBUF):
        wait_out(g0 + b, b)


def _make_kernel(n_idx):
    mesh = plsc.VectorSubcoreMesh(core_axis_name="c", subcore_axis_name="s")
    per_w = n_idx // NW
    return pl.kernel(
        _body,
        mesh=mesh,
        out_type=jax.ShapeDtypeStruct((n_idx, EMB), jnp.float32),
        scratch_types=[
            pltpu.VMEM_SHARED((PAD_SEG, EMB), jnp.float32),
            pltpu.VMEM((per_w,), jnp.int32),
            pltpu.VMEM((NBUF, CHUNK, EMB), jnp.float32),
            pltpu.VMEM((ROWS_PER_SUB, EMB), jnp.float32),
            pltpu.SemaphoreType.DMA((NBUF,)),
            pltpu.SemaphoreType.DMA((NBUF,)),
        ],
    )


@jax.jit
def kernel(word, seg_embedding_weight):
    batch, seq = word.shape
    word_flat = word.reshape(-1).astype(jnp.int32)
    table_pad = jnp.zeros((PAD_SEG, EMB), jnp.float32).at[:SEG, :].set(
        seg_embedding_weight)
    out = _make_kernel(batch * seq)(word_flat, table_pad)
    out = out.reshape(batch, seq, EMB)
    return with_layout_constraint(out, Layout(major_to_minor=(0, 1, 2)))
